# baseline (speedup 1.0000x reference)
import functools

import jax
import jax.numpy as jnp
from jax import lax
from jax.experimental import pallas as pl
from jax.experimental.pallas import tpu as pltpu
from jax.experimental.pallas import tpu_sc as plsc

_B = 1024
_S = 200
_E = 64
_R = 1000
_RP = 1024

_HALF = 100
_NW = 32
_ROWS_W = _B // _NW
_HALVES_W = _ROWS_W * 2


@functools.partial(jax.jit, static_argnames=())
def _sc_gather_sum(table, ids2):
    mesh = plsc.VectorSubcoreMesh(core_axis_name="c", subcore_axis_name="s")

    @functools.partial(
        pl.kernel,
        out_type=(
            jax.ShapeDtypeStruct((2 * _B, _HALF, _E), jnp.float32),
            jax.ShapeDtypeStruct((_B, _E), jnp.float32),
        ),
        mesh=mesh,
        scratch_types=[
            pltpu.VMEM((_HALVES_W, _HALF), jnp.int32),
            pltpu.VMEM((_HALF, _E), jnp.float32),
            pltpu.VMEM((_HALF, _E), jnp.float32),
            pltpu.VMEM((_ROWS_W, _E), jnp.float32),
            pltpu.SemaphoreType.DMA,
            pltpu.SemaphoreType.DMA,
        ],
        compiler_params=pltpu.CompilerParams(use_tc_tiling_on_sc=False),
    )
    def k(table_hbm, ids_hbm, emb_out, sums_out, idx_v, buf0, buf1, sums_v,
          sem0, sem1):
        cid = lax.axis_index("c")
        sid = lax.axis_index("s")
        wid = sid * 2 + cid
        base_half = wid * _HALVES_W
        base_row = wid * _ROWS_W

        pltpu.sync_copy(ids_hbm.at[pl.ds(base_half, _HALVES_W)], idx_v)

        zero = jnp.zeros((16,), jnp.float32)

        def acc_half(buf, acc):
            @pl.loop(0, _HALF, init_carry=acc, unroll=4)
            def inner(s, acc):
                a0, a1, a2, a3 = acc
                return (
                    a0 + buf[s, pl.ds(0, 16)],
                    a1 + buf[s, pl.ds(16, 16)],
                    a2 + buf[s, pl.ds(32, 16)],
                    a3 + buf[s, pl.ds(48, 16)],
                )
            return inner

        @pl.loop(0, _ROWS_W)
        def row_loop(r):
            h0 = base_half + 2 * r
            cp0 = pltpu.async_copy(table_hbm.at[idx_v.at[2 * r]], buf0, sem0)
            cp1 = pltpu.async_copy(table_hbm.at[idx_v.at[2 * r + 1]], buf1, sem1)
            cp0.wait()
            acc = acc_half(buf0, (zero, zero, zero, zero))
            pltpu.sync_copy(buf0, emb_out.at[h0])
            cp1.wait()
            acc = acc_half(buf1, acc)
            pltpu.sync_copy(buf1, emb_out.at[h0 + 1])
            sums_v[r, pl.ds(0, 16)] = acc[0]
            sums_v[r, pl.ds(16, 16)] = acc[1]
            sums_v[r, pl.ds(32, 16)] = acc[2]
            sums_v[r, pl.ds(48, 16)] = acc[3]

        pltpu.sync_copy(sums_v, sums_out.at[pl.ds(base_row, _ROWS_W)])

    return k(table, ids2)


def _mlp_body(s_ref, w1_ref, b1_ref, w2_ref, b2_ref, o_ref):
    x = s_ref[...] * (1.0 / _S)
    h = lax.dot_general(x, w1_ref[...], (((1,), (1,)), ((), ())),
                        preferred_element_type=jnp.float32) + b1_ref[...]
    h = jnp.maximum(h, 0.0)
    o_ref[...] = lax.dot_general(h, w2_ref[...], (((1,), (1,)), ((), ())),
                                 preferred_element_type=jnp.float32) + b2_ref[...]


def _mlp(sums, W1, b1, W2p, b2p):
    return pl.pallas_call(
        _mlp_body,
        out_shape=jax.ShapeDtypeStruct((_B, _RP), jnp.float32),
    )(sums, W1, b1.reshape(1, _E), W2p, b2p.reshape(1, _RP))


def kernel(atom_ids, atom_table, W1, b1, W2, b2):
    ids2 = atom_ids.astype(jnp.int32).reshape(2 * _B, _HALF)
    emb2, sums = _sc_gather_sum(atom_table, ids2)
    embeddings = emb2.reshape(_B, _S, _E)
    W2p = jnp.pad(W2, ((0, _RP - _R), (0, 0)))
    b2p = jnp.pad(b2, (0, _RP - _R))
    scores = _mlp(sums, W1, b1, W2p, b2p)[:, :_R]
    return scores, embeddings


# speedup vs baseline: 1.0418x; 1.0418x over previous
import functools

import jax
import jax.numpy as jnp
from jax import lax
from jax.experimental import pallas as pl
from jax.experimental.pallas import tpu as pltpu
from jax.experimental.pallas import tpu_sc as plsc

_B = 1024
_S = 200
_E = 64
_R = 1000
_RP = 1024

_HALF = 100
_NW = 32
_ROWS_W = _B // _NW
_HALVES_W = _ROWS_W * 2


@functools.partial(jax.jit, static_argnames=())
def _sc_gather_sum(table, ids2):
    mesh = plsc.VectorSubcoreMesh(core_axis_name="c", subcore_axis_name="s")

    nbuf = 8
    dist = 4

    @functools.partial(
        pl.kernel,
        out_type=(
            jax.ShapeDtypeStruct((2 * _B, _HALF, _E), jnp.float32),
            jax.ShapeDtypeStruct((_B, _E), jnp.float32),
        ),
        mesh=mesh,
        scratch_types=[
            pltpu.VMEM((_HALVES_W, _HALF), jnp.int32),
            [pltpu.VMEM((_HALF, _E), jnp.float32) for _ in range(nbuf)],
            pltpu.VMEM((_ROWS_W, _E), jnp.float32),
            [pltpu.SemaphoreType.DMA for _ in range(nbuf)],
            [pltpu.SemaphoreType.DMA for _ in range(nbuf)],
        ],
        compiler_params=pltpu.CompilerParams(use_tc_tiling_on_sc=False),
    )
    def k(table_hbm, ids_hbm, emb_out, sums_out, idx_v, bufs, sums_v,
          gsems, ssems):
        cid = lax.axis_index("c")
        sid = lax.axis_index("s")
        wid = sid * 2 + cid
        base_half = wid * _HALVES_W
        base_row = wid * _ROWS_W

        pltpu.sync_copy(ids_hbm.at[pl.ds(base_half, _HALVES_W)], idx_v)

        zero = jnp.zeros((16,), jnp.float32)

        def fire_gather(h, b):
            pltpu.async_copy(table_hbm.at[idx_v.at[h]], bufs[b], gsems[b])

        def acc_half(buf, acc):
            @pl.loop(0, _HALF, init_carry=acc, unroll=4)
            def inner(s, acc):
                a0, a1, a2, a3 = acc
                return (
                    a0 + buf[s, pl.ds(0, 16)],
                    a1 + buf[s, pl.ds(16, 16)],
                    a2 + buf[s, pl.ds(32, 16)],
                    a3 + buf[s, pl.ds(48, 16)],
                )
            return inner

        for b in range(dist):
            fire_gather(b, b)

        @pl.loop(0, _HALVES_W, step=nbuf)
        def outer(g0):
            acc = None
            for b in range(nbuf):
                g = g0 + b
                pltpu.make_async_copy(
                    table_hbm.at[idx_v.at[g]], bufs[b], gsems[b]).wait()
                if b % 2 == 0:
                    acc = acc_half(bufs[b], (zero, zero, zero, zero))
                else:
                    acc = acc_half(bufs[b], acc)
                    r = (g0 + b - 1) // 2
                    sums_v[r, pl.ds(0, 16)] = acc[0]
                    sums_v[r, pl.ds(16, 16)] = acc[1]
                    sums_v[r, pl.ds(32, 16)] = acc[2]
                    sums_v[r, pl.ds(48, 16)] = acc[3]
                pltpu.async_copy(bufs[b], emb_out.at[base_half + g], ssems[b])
                b2 = (b + dist) % nbuf
                g2 = g + dist

                @pl.when(g2 - nbuf >= 0)
                def _():
                    pltpu.make_async_copy(
                        bufs[b2], emb_out.at[base_half + g2 - nbuf],
                        ssems[b2]).wait()

                @pl.when(g2 < _HALVES_W)
                def _():
                    fire_gather(g2, b2)

        for i in range(dist):
            g = _HALVES_W - dist + i
            b = g % nbuf
            pltpu.make_async_copy(
                bufs[b], emb_out.at[base_half + g], ssems[b]).wait()

        pltpu.sync_copy(sums_v, sums_out.at[pl.ds(base_row, _ROWS_W)])

    return k(table, ids2)


def _mlp_body(s_ref, w1_ref, b1_ref, w2_ref, b2_ref, o_ref):
    x = s_ref[...] * (1.0 / _S)
    h = lax.dot_general(x, w1_ref[...], (((1,), (1,)), ((), ())),
                        preferred_element_type=jnp.float32) + b1_ref[...]
    h = jnp.maximum(h, 0.0)
    o_ref[...] = lax.dot_general(h, w2_ref[...], (((1,), (1,)), ((), ())),
                                 preferred_element_type=jnp.float32) + b2_ref[...]


def _mlp(sums, W1, b1, W2p, b2p):
    return pl.pallas_call(
        _mlp_body,
        out_shape=jax.ShapeDtypeStruct((_B, _RP), jnp.float32),
    )(sums, W1, b1.reshape(1, _E), W2p, b2p.reshape(1, _RP))


def kernel(atom_ids, atom_table, W1, b1, W2, b2):
    ids2 = atom_ids.astype(jnp.int32).reshape(2 * _B, _HALF)
    emb2, sums = _sc_gather_sum(atom_table, ids2)
    embeddings = emb2.reshape(_B, _S, _E)
    W2p = jnp.pad(W2, ((0, _RP - _R), (0, 0)))
    b2p = jnp.pad(b2, (0, _RP - _R))
    scores = _mlp(sums, W1, b1, W2p, b2p)[:, :_R]
    return scores, embeddings
